# baseline (speedup 1.0000x reference)
import jax
import jax.numpy as jnp
from jax import lax
from jax.experimental import pallas as pl
from jax.experimental.pallas import tpu as pltpu
from jax.experimental.pallas import tpu_sc as plsc

D = 64
NC, NS, L = 2, 16, 16
NW = NC * NS

B = 4096
C = 200
RB = B // 128
N_ITEMS = C * RB
ITEMS_PER_W = N_ITEMS // NW


def _transpose_block(g, idx_v, t, rowv, midv):
    def body(i, c2):
        r0 = i * 8
        for u in range(8):
            r = r0 + u
            colv = lax.broadcast(r, (L,))
            for k in range(4):
                v = g[r, pl.ds(k * L, L)]
                plsc.store_scatter(t, [rowv[k], midv, colv], v)
        return c2

    lax.fori_loop(0, 16, body, 0, unroll=False)

    def mred(j, acc):
        return acc | (idx_v[pl.ds(j * L, L)] == 0)

    mv = lax.fori_loop(1, 8, mred, idx_v[pl.ds(0, L)] == 0)
    nz = plsc.all_reduce_population_count(mv)[0]

    @pl.when(nz > 0)
    def _fixup():
        zeros = jnp.zeros((L,), jnp.float32)

        def fix_group(j, c3):
            m = idx_v[pl.ds(j * L, L)] == 0
            rv = j * L + lax.iota(jnp.int32, L)

            def fh(h, c4):
                rows = lax.broadcast(h // 8, (L,))
                mids = lax.broadcast(h % 8, (L,))
                plsc.store_scatter(t, [rows, mids, rv], zeros, mask=m)
                return c4

            lax.fori_loop(0, D, fh, 0)
            return c3

        lax.fori_loop(0, 8, fix_group, 0)


def _emb_body(table_hbm, idx_hbm, out_hbm,
              idx0, idx1, g0, g1, t0, t1,
              isem0, isem1, gsem0, gsem1, wsem0, wsem1):
    wid = lax.axis_index("s") * NC + lax.axis_index("c")
    m0 = wid * ITEMS_PER_W

    idx_bufs = (idx0, idx1)
    g_bufs = (g0, g1)
    t_bufs = (t0, t1)
    isems = (isem0, isem1)
    gsems = (gsem0, gsem1)
    wsems = (wsem0, wsem1)

    lane = lax.iota(jnp.int32, L)
    midv = lane % 8
    rowv = tuple(2 * k + (lane >= 8).astype(jnp.int32) for k in range(4))

    def idx_src(j):
        m = m0 + j
        c = m // RB
        rb = m % RB
        return idx_hbm.at[pl.ds(c * B + rb * 128, 128)]

    def out_dst(j):
        m = m0 + j
        c = m // RB
        rb = m % RB
        return out_hbm.at[c, :, rb]

    pltpu.async_copy(idx_src(0), idx0, isem0)
    pltpu.async_copy(idx_src(1), idx1, isem1)
    pltpu.make_async_copy(idx_src(0), idx0, isem0).wait()
    pltpu.async_copy(table_hbm.at[idx0], g0, gsem0)

    def phase(j, p):
        q = 1 - p

        pltpu.make_async_copy(table_hbm.at[idx_bufs[p]], g_bufs[p],
                              gsems[p]).wait()

        @pl.when(j + 2 < ITEMS_PER_W)
        def _pref():
            pltpu.async_copy(idx_src(j + 2), idx_bufs[p], isems[p])

        @pl.when(j + 1 < ITEMS_PER_W)
        def _next():
            pltpu.make_async_copy(idx_src(j + 1), idx_bufs[q],
                                  isems[q]).wait()
            pltpu.async_copy(table_hbm.at[idx_bufs[q]], g_bufs[q],
                             gsems[q])

        @pl.when(j >= 2)
        def _drain():
            pltpu.make_async_copy(t_bufs[p], out_dst(j - 2),
                                  wsems[p]).wait()

        _transpose_block(g_bufs[p], idx_bufs[p], t_bufs[p], rowv, midv)

        pltpu.async_copy(t_bufs[p], out_dst(j), wsems[p])

    def body(i, carry):
        phase(i * 2, 0)
        phase(i * 2 + 1, 1)
        return carry

    lax.fori_loop(0, ITEMS_PER_W // 2, body, 0)

    pltpu.make_async_copy(t0, out_dst(ITEMS_PER_W - 2), wsems[0]).wait()
    pltpu.make_async_copy(t1, out_dst(ITEMS_PER_W - 1), wsems[1]).wait()


@jax.jit
def _emb(idx_t, table):
    mesh = plsc.VectorSubcoreMesh(core_axis_name="c", subcore_axis_name="s")
    f = pl.kernel(
        _emb_body,
        out_type=jax.ShapeDtypeStruct((C, 8, RB, 8, 128), jnp.float32),
        mesh=mesh,
        compiler_params=pltpu.CompilerParams(needs_layout_passes=False,
                                             use_tc_tiling_on_sc=False),
        scratch_types=[
            pltpu.VMEM((128,), jnp.int32),
            pltpu.VMEM((128,), jnp.int32),
            pltpu.VMEM((128, D), jnp.float32),
            pltpu.VMEM((128, D), jnp.float32),
            pltpu.VMEM((8, 8, 128), jnp.float32),
            pltpu.VMEM((8, 8, 128), jnp.float32),
            pltpu.SemaphoreType.DMA,
            pltpu.SemaphoreType.DMA,
            pltpu.SemaphoreType.DMA,
            pltpu.SemaphoreType.DMA,
            pltpu.SemaphoreType.DMA,
            pltpu.SemaphoreType.DMA,
        ],
    )
    return f(table, idx_t)


def kernel(inputs, shared_weights):
    idx_t = inputs.T.reshape(-1).astype(jnp.int32)
    out5 = _emb(idx_t, shared_weights)
    return out5.transpose(2, 4, 0, 1, 3).reshape(B, C, D)
